# baseline (speedup 1.0000x reference)
import functools

import jax
import jax.numpy as jnp
from jax.experimental import pallas as pl

_K = 3
_BASE = 4096
_SHIFT = 1024
_BR = 2000


def _voxel_key(v):
    v = v.astype(jnp.int32) + _SHIFT
    return (v[:, 0] * _BASE + v[:, 1]) * _BASE + v[:, 2]


def _mm_body(f_ref, w_ref, p_ref):
    p_ref[0] = jnp.dot(f_ref[...], w_ref[0], preferred_element_type=jnp.float32)


@functools.partial(jax.jit, static_argnames=())
def _matmul27(features, W):
    n_in = features.shape[0]
    nblk = n_in // _BR
    return pl.pallas_call(
        _mm_body,
        grid=(nblk, _K * _K * _K),
        in_specs=[
            pl.BlockSpec((_BR, 128), lambda i, r: (i, 0)),
            pl.BlockSpec((1, 128, 128), lambda i, r: (r, 0, 0)),
        ],
        out_specs=pl.BlockSpec((1, _BR, 128), lambda i, r: (r, i, 0)),
        out_shape=jax.ShapeDtypeStruct((_K * _K * _K, n_in, 128), jnp.float32),
    )(features, W)


def kernel(features, inp_positions, out_positions, W):
    n_out = out_positions.shape[0]
    vin = jnp.floor(inp_positions).astype(jnp.int32)
    vout = jnp.floor(out_positions).astype(jnp.int32)
    inkey = _voxel_key(vin)
    outkey = _voxel_key(vout)
    order = jnp.argsort(outkey)
    sorted_keys = outkey[order]

    half = _K // 2
    offs = []
    for dx in range(_K):
        for dy in range(_K):
            for dz in range(_K):
                offs.append(((dx - half) * _BASE + (dy - half)) * _BASE + (dz - half))
    offs = jnp.array(offs, dtype=jnp.int32)

    tkey = inkey[None, :] + offs[:, None]
    pos = jnp.searchsorted(sorted_keys, tkey.ravel()).reshape(tkey.shape)
    pos_c = jnp.clip(pos, 0, n_out - 1)
    valid = sorted_keys[pos_c] == tkey
    dst = jnp.where(valid, order[pos_c], n_out)

    P = _matmul27(features, W)
    out = jnp.zeros((n_out, W.shape[2]), dtype=features.dtype)
    out = out.at[dst.ravel()].add(P.reshape(-1, 128))
    return out


# speedup vs baseline: 1.0116x; 1.0116x over previous
import functools

import jax
import jax.numpy as jnp
from jax import lax
from jax.experimental import pallas as pl
from jax.experimental.pallas import tpu as pltpu
from jax.experimental.pallas import tpu_sc as plsc

_K = 3
_BASE = 4096
_SHIFT = 1024
_BR = 2000
_NOFF = _K * _K * _K

_NSUB = 16
_N_IN = 50000
_N_OUT = 100000
_NPAIR = _NOFF * _N_IN
_SEG = 1024
_SEGS = 83
_CHUNKS = _SEG // 128
_NPAD = _NSUB * _SEGS * _SEG
_ACCR = 100352
_STRIPE = _ACCR // _NSUB
_DUMP = _N_OUT
_NP16 = _NPAIR * 8


def _voxel_key(v):
    v = v.astype(jnp.int32) + _SHIFT
    return (v[:, 0] * _BASE + v[:, 1]) * _BASE + v[:, 2]


def _mm_body(f_ref, w_ref, p_ref):
    p_ref[0] = jnp.dot(f_ref[...], w_ref[0], preferred_element_type=jnp.float32)


def _matmul27(features, W):
    n_in = features.shape[0]
    nblk = n_in // _BR
    return pl.pallas_call(
        _mm_body,
        grid=(nblk, _NOFF),
        in_specs=[
            pl.BlockSpec((_BR, 128), lambda i, r: (i, 0)),
            pl.BlockSpec((1, 128, 128), lambda i, r: (r, 0, 0)),
        ],
        out_specs=pl.BlockSpec((1, _BR, 128), lambda i, r: (r, i, 0)),
        out_shape=jax.ShapeDtypeStruct((_NOFF, n_in, 128), jnp.float32),
    )(features, W)


def _sc_body(dst2_hbm, p16_hbm, out8_hbm, dseg, pidx, rowbuf, zbuf, acc, sem):
    c = lax.axis_index("c")
    s = lax.axis_index("s")
    ivec = lax.iota(jnp.int32, 16)
    zeros16 = jnp.zeros((16,), jnp.float32)

    def _zb(i, carry):
        zbuf[i] = zeros16
        return carry
    lax.fori_loop(0, 128, _zb, 0)

    for kk in range(4):
        k = c * 4 + kk

        def _z(j, carry):
            pltpu.sync_copy(zbuf, acc.at[pl.ds(
                pl.multiple_of(s * _STRIPE + j * 128, 128), 128)])
            return carry
        lax.fori_loop(0, _STRIPE // 128, _z, 0)
        plsc.subcore_barrier()

        def _seg(seg, carry):
            segbase = (s * _SEGS + seg) * _SEG
            pltpu.sync_copy(dst2_hbm.at[pl.ds(
                pl.multiple_of(segbase // 128, _CHUNKS), _CHUNKS)], dseg)

            def _pb(j, carry2):
                for t in range(8):
                    pair = segbase + j * 128 + t * 16 + ivec
                    pidx[j, pl.ds(t * 16, 16)] = jnp.minimum(
                        pair * 8 + k, _NP16 - 1)
                return carry2
            lax.fori_loop(0, _CHUNKS, _pb, 0)

            copies = [
                pltpu.async_copy(p16_hbm.at[pidx.at[j]],
                                 rowbuf.at[pl.ds(j * 128, 128)], sem)
                for j in range(_CHUNKS)
            ]
            for cp in copies:
                cp.wait()
            for j in range(_CHUNKS):
                pltpu.sync_copy(rowbuf.at[pl.ds(j * 128, 128)],
                                acc.at[dseg.at[j]], add=True)
            return carry
        lax.fori_loop(0, _SEGS, _seg, 0)
        plsc.subcore_barrier()

        pltpu.sync_copy(
            acc.at[pl.ds(pl.multiple_of(s * _STRIPE, _STRIPE), _STRIPE)],
            out8_hbm.at[pl.ds(
                pl.multiple_of(k * _ACCR + s * _STRIPE, _STRIPE), _STRIPE)])


@functools.partial(
    pl.kernel,
    out_type=jax.ShapeDtypeStruct((8 * _ACCR, 16), jnp.float32),
    mesh=plsc.VectorSubcoreMesh(core_axis_name="c", subcore_axis_name="s"),
    compiler_params=pltpu.CompilerParams(use_tc_tiling_on_sc=False),
    scratch_types=[
        pltpu.VMEM((_CHUNKS, 128), jnp.int32),
        pltpu.VMEM((_CHUNKS, 128), jnp.int32),
        pltpu.VMEM((_SEG, 16), jnp.float32),
        pltpu.VMEM((128, 16), jnp.float32),
        pltpu.VMEM_SHARED((_ACCR, 16), jnp.float32),
        pltpu.SemaphoreType.DMA,
    ],
)
def _sc_accum(dst2_hbm, p16_hbm, out8_hbm, *scratch):
    _sc_body(dst2_hbm, p16_hbm, out8_hbm, *scratch)


def kernel(features, inp_positions, out_positions, W):
    n_out = out_positions.shape[0]
    vin = jnp.floor(inp_positions).astype(jnp.int32)
    vout = jnp.floor(out_positions).astype(jnp.int32)
    inkey = _voxel_key(vin)
    outkey = _voxel_key(vout)
    order = jnp.argsort(outkey)
    sorted_keys = outkey[order]

    half = _K // 2
    offs = []
    for dx in range(_K):
        for dy in range(_K):
            for dz in range(_K):
                offs.append(((dx - half) * _BASE + (dy - half)) * _BASE + (dz - half))
    offs = jnp.array(offs, dtype=jnp.int32)

    tkey = inkey[None, :] + offs[:, None]
    pos = jnp.searchsorted(sorted_keys, tkey.ravel()).reshape(tkey.shape)
    pos_c = jnp.clip(pos, 0, n_out - 1)
    valid = sorted_keys[pos_c] == tkey
    dst = jnp.where(valid, order[pos_c], _DUMP)

    dst_pad = jnp.full((_NPAD,), _DUMP, jnp.int32)
    dst_pad = lax.dynamic_update_slice(dst_pad, dst.ravel(), (0,))

    P = _matmul27(features, W)
    p16 = P.reshape(_NP16, 16)
    out8 = _sc_accum(dst_pad.reshape(-1, 128), p16)
    out = (out8.reshape(8, _ACCR, 16)[:, :n_out]
           .transpose(1, 0, 2).reshape(n_out, 128))
    return out


# speedup vs baseline: 7.5519x; 7.4652x over previous
import functools

import jax
import jax.numpy as jnp
from jax import lax
from jax.experimental import pallas as pl
from jax.experimental.pallas import tpu as pltpu
from jax.experimental.pallas import tpu_sc as plsc

_K = 3
_BASE = 4096
_SHIFT = 1024
_BR = 2000
_NOFF = _K * _K * _K

_NSUB = 16
_N_IN = 50000
_N_OUT = 100000
_NPAIR = _NOFF * _N_IN
_SEG = 1024
_SEGS = 83
_CHUNKS = _SEG // 128
_NPAD = _NSUB * _SEGS * _SEG
_ACCR = 100352
_STRIPE = _ACCR // _NSUB
_DUMP = _N_OUT
_NP16 = _NPAIR * 8


def _voxel_key(v):
    v = v.astype(jnp.int32) + _SHIFT
    return (v[:, 0] * _BASE + v[:, 1]) * _BASE + v[:, 2]


def _mm_body(f_ref, w_ref, p_ref):
    p_ref[0] = jnp.dot(f_ref[...], w_ref[0], preferred_element_type=jnp.float32)


def _matmul27(features, W):
    n_in = features.shape[0]
    nblk = n_in // _BR
    return pl.pallas_call(
        _mm_body,
        grid=(nblk, _NOFF),
        in_specs=[
            pl.BlockSpec((_BR, 128), lambda i, r: (i, 0)),
            pl.BlockSpec((1, 128, 128), lambda i, r: (r, 0, 0)),
        ],
        out_specs=pl.BlockSpec((1, _BR, 128), lambda i, r: (r, i, 0)),
        out_shape=jax.ShapeDtypeStruct((_NOFF, n_in, 128), jnp.float32),
    )(features, W)


def _sc_body(dst2_hbm, p16_hbm, out8_hbm, dseg, pidx, rowbuf, zbuf, acc, sem):
    c = lax.axis_index("c")
    s = lax.axis_index("s")
    ivec = lax.iota(jnp.int32, 16)
    zeros16 = jnp.zeros((16,), jnp.float32)

    def _zb(i, carry):
        zbuf[i] = zeros16
        return carry
    lax.fori_loop(0, 128, _zb, 0)

    for kk in range(4):
        k = c * 4 + kk

        def _z(j, carry):
            pltpu.sync_copy(zbuf, acc.at[pl.ds(
                pl.multiple_of(s * _STRIPE + j * 128, 128), 128)])
            return carry
        lax.fori_loop(0, _STRIPE // 128, _z, 0)
        plsc.subcore_barrier()

        def _seg(seg, carry):
            segbase = (s * _SEGS + seg) * _SEG
            pltpu.sync_copy(dst2_hbm.at[pl.ds(
                pl.multiple_of(segbase // 128, _CHUNKS), _CHUNKS)], dseg)

            def _pb(j, carry2):
                for t in range(8):
                    pair = segbase + j * 128 + t * 16 + ivec
                    pidx[j, pl.ds(t * 16, 16)] = jnp.minimum(
                        pair * 8 + k, _NP16 - 1)
                return carry2
            lax.fori_loop(0, _CHUNKS, _pb, 0)

            copies = [
                pltpu.async_copy(p16_hbm.at[pidx.at[j]],
                                 rowbuf.at[pl.ds(j * 128, 128)], sem)
                for j in range(_CHUNKS)
            ]
            for cp in copies:
                cp.wait()
            for j in range(_CHUNKS):
                pltpu.sync_copy(rowbuf.at[pl.ds(j * 128, 128)],
                                acc.at[dseg.at[j]], add=True)
            return carry
        lax.fori_loop(0, _SEGS, _seg, 0)
        plsc.subcore_barrier()

        pltpu.sync_copy(
            acc.at[pl.ds(pl.multiple_of(s * _STRIPE, _STRIPE), _STRIPE)],
            out8_hbm.at[pl.ds(
                pl.multiple_of(k * _ACCR + s * _STRIPE, _STRIPE), _STRIPE)])


@functools.partial(
    pl.kernel,
    out_type=jax.ShapeDtypeStruct((8 * _ACCR, 16), jnp.float32),
    mesh=plsc.VectorSubcoreMesh(core_axis_name="c", subcore_axis_name="s"),
    compiler_params=pltpu.CompilerParams(use_tc_tiling_on_sc=False),
    scratch_types=[
        pltpu.VMEM((_CHUNKS, 128), jnp.int32),
        pltpu.VMEM((_CHUNKS, 128), jnp.int32),
        pltpu.VMEM((_SEG, 16), jnp.float32),
        pltpu.VMEM((128, 16), jnp.float32),
        pltpu.VMEM_SHARED((_ACCR, 16), jnp.float32),
        pltpu.SemaphoreType.DMA,
    ],
)
def _sc_accum(dst2_hbm, p16_hbm, out8_hbm, *scratch):
    _sc_body(dst2_hbm, p16_hbm, out8_hbm, *scratch)


def kernel(features, inp_positions, out_positions, W):
    n_out = out_positions.shape[0]
    vin = jnp.floor(inp_positions).astype(jnp.int32)
    vout = jnp.floor(out_positions).astype(jnp.int32)
    inkey = _voxel_key(vin)
    outkey = _voxel_key(vout)
    order = jnp.argsort(outkey)
    sorted_keys = outkey[order]

    half = _K // 2
    offs = []
    for dx in range(_K):
        for dy in range(_K):
            for dz in range(_K):
                offs.append(((dx - half) * _BASE + (dy - half)) * _BASE + (dz - half))
    dsts = []
    for off in offs:
        tkey = inkey + jnp.int32(off)
        pos = jnp.searchsorted(sorted_keys, tkey)
        pos_c = jnp.clip(pos, 0, n_out - 1)
        valid = sorted_keys[pos_c] == tkey
        dsts.append(jnp.where(valid, order[pos_c], _DUMP))
    dst = jnp.stack(dsts)

    dst_pad = jnp.full((_NPAD,), _DUMP, jnp.int32)
    dst_pad = lax.dynamic_update_slice(dst_pad, dst.ravel(), (0,))

    P = _matmul27(features, W)
    p16 = P.reshape(_NP16, 16)
    out8 = _sc_accum(dst_pad.reshape(-1, 128), p16)
    out = (out8.reshape(8, _ACCR, 16)[:, :n_out]
           .transpose(1, 0, 2).reshape(n_out, 128))
    return out


# speedup vs baseline: 7.5622x; 1.0014x over previous
import functools

import jax
import jax.numpy as jnp
from jax import lax
from jax.experimental import pallas as pl
from jax.experimental.pallas import tpu as pltpu
from jax.experimental.pallas import tpu_sc as plsc

_K = 3
_BASE = 4096
_SHIFT = 1024
_BR = 2000
_NOFF = _K * _K * _K

_NSUB = 16
_N_IN = 50000
_N_OUT = 100000
_NPAIR = _NOFF * _N_IN
_SEG = 512
_SEGS = 166
_CHUNKS = _SEG // 128
_NPAD = _NSUB * _SEGS * _SEG
_ACCR = 100352
_STRIPE = _ACCR // _NSUB
_DUMP = _N_OUT
_NP16 = _NPAIR * 8


def _voxel_key(v):
    v = v.astype(jnp.int32) + _SHIFT
    return (v[:, 0] * _BASE + v[:, 1]) * _BASE + v[:, 2]


def _mm_body(f_ref, w_ref, p_ref):
    p_ref[0] = jnp.dot(f_ref[...], w_ref[0], preferred_element_type=jnp.float32)


def _matmul27(features, W):
    n_in = features.shape[0]
    nblk = n_in // _BR
    return pl.pallas_call(
        _mm_body,
        grid=(nblk, _NOFF),
        in_specs=[
            pl.BlockSpec((_BR, 128), lambda i, r: (i, 0)),
            pl.BlockSpec((1, 128, 128), lambda i, r: (r, 0, 0)),
        ],
        out_specs=pl.BlockSpec((1, _BR, 128), lambda i, r: (r, i, 0)),
        out_shape=jax.ShapeDtypeStruct((_NOFF, n_in, 128), jnp.float32),
    )(features, W)


def _sc_body(dst2_hbm, p16_hbm, out8_hbm, dseg0, dseg1, pidx0, pidx1,
             rowbuf0, rowbuf1, zbuf, acc, semg0, semg1, semd0, semd1):
    c = lax.axis_index("c")
    s = lax.axis_index("s")
    ivec = lax.iota(jnp.int32, 16)
    zeros16 = jnp.zeros((16,), jnp.float32)
    dsegs = (dseg0, dseg1)
    pidxs = (pidx0, pidx1)
    rowbufs = (rowbuf0, rowbuf1)
    semgs = (semg0, semg1)
    semds = (semd0, semd1)

    def _zb(i, carry):
        zbuf[i] = zeros16
        return carry
    lax.fori_loop(0, 128, _zb, 0)

    def _start(seg, k, b):
        segbase = (s * _SEGS + seg) * _SEG
        pltpu.async_copy(dst2_hbm.at[pl.ds(
            pl.multiple_of(segbase // 128, _CHUNKS), _CHUNKS)],
            dsegs[b], semds[b])

        def _pb(j, carry2):
            for t in range(8):
                pair = segbase + j * 128 + t * 16 + ivec
                pidxs[b][j, pl.ds(t * 16, 16)] = jnp.minimum(
                    pair * 8 + k, _NP16 - 1)
            return carry2
        lax.fori_loop(0, _CHUNKS, _pb, 0)
        for j in range(_CHUNKS):
            pltpu.async_copy(p16_hbm.at[pidxs[b].at[j]],
                             rowbufs[b].at[pl.ds(j * 128, 128)], semgs[b])

    def _drain(b):
        pltpu.make_async_copy(dst2_hbm.at[pl.ds(0, _CHUNKS)],
                              dsegs[b], semds[b]).wait()
        for j in range(_CHUNKS):
            pltpu.make_async_copy(p16_hbm.at[pidxs[b].at[j]],
                                  rowbufs[b].at[pl.ds(j * 128, 128)],
                                  semgs[b]).wait()
        for j in range(_CHUNKS):
            pltpu.sync_copy(rowbufs[b].at[pl.ds(j * 128, 128)],
                            acc.at[dsegs[b].at[j]], add=True)

    for kk in range(4):
        k = c * 4 + kk

        def _z(j, carry):
            pltpu.sync_copy(zbuf, acc.at[pl.ds(
                pl.multiple_of(s * _STRIPE + j * 128, 128), 128)])
            return carry
        lax.fori_loop(0, _STRIPE // 128, _z, 0)
        plsc.subcore_barrier()

        _start(0, k, 0)

        def _segpair(sp, carry):
            _start(2 * sp + 1, k, 1)
            _drain(0)

            @pl.when(2 * sp + 2 < _SEGS)
            def _():
                _start(2 * sp + 2, k, 0)
            _drain(1)
            return carry
        lax.fori_loop(0, _SEGS // 2, _segpair, 0)
        plsc.subcore_barrier()

        pltpu.sync_copy(
            acc.at[pl.ds(pl.multiple_of(s * _STRIPE, _STRIPE), _STRIPE)],
            out8_hbm.at[pl.ds(
                pl.multiple_of(k * _ACCR + s * _STRIPE, _STRIPE), _STRIPE)])


@functools.partial(
    pl.kernel,
    out_type=jax.ShapeDtypeStruct((8 * _ACCR, 16), jnp.float32),
    mesh=plsc.VectorSubcoreMesh(core_axis_name="c", subcore_axis_name="s"),
    compiler_params=pltpu.CompilerParams(use_tc_tiling_on_sc=False),
    scratch_types=[
        pltpu.VMEM((_CHUNKS, 128), jnp.int32),
        pltpu.VMEM((_CHUNKS, 128), jnp.int32),
        pltpu.VMEM((_CHUNKS, 128), jnp.int32),
        pltpu.VMEM((_CHUNKS, 128), jnp.int32),
        pltpu.VMEM((_SEG, 16), jnp.float32),
        pltpu.VMEM((_SEG, 16), jnp.float32),
        pltpu.VMEM((128, 16), jnp.float32),
        pltpu.VMEM_SHARED((_ACCR, 16), jnp.float32),
        pltpu.SemaphoreType.DMA,
        pltpu.SemaphoreType.DMA,
        pltpu.SemaphoreType.DMA,
        pltpu.SemaphoreType.DMA,
    ],
)
def _sc_accum(dst2_hbm, p16_hbm, out8_hbm, *scratch):
    _sc_body(dst2_hbm, p16_hbm, out8_hbm, *scratch)


def kernel(features, inp_positions, out_positions, W):
    n_out = out_positions.shape[0]
    vin = jnp.floor(inp_positions).astype(jnp.int32)
    vout = jnp.floor(out_positions).astype(jnp.int32)
    inkey = _voxel_key(vin)
    outkey = _voxel_key(vout)
    order = jnp.argsort(outkey)
    sorted_keys = outkey[order]

    half = _K // 2
    offs = []
    for dx in range(_K):
        for dy in range(_K):
            for dz in range(_K):
                offs.append(((dx - half) * _BASE + (dy - half)) * _BASE + (dz - half))
    dsts = []
    for off in offs:
        tkey = inkey + jnp.int32(off)
        pos = jnp.searchsorted(sorted_keys, tkey)
        pos_c = jnp.clip(pos, 0, n_out - 1)
        valid = sorted_keys[pos_c] == tkey
        dsts.append(jnp.where(valid, order[pos_c], _DUMP))
    dst = jnp.stack(dsts)

    dst_pad = jnp.full((_NPAD,), _DUMP, jnp.int32)
    dst_pad = lax.dynamic_update_slice(dst_pad, dst.ravel(), (0,))

    P = _matmul27(features, W)
    p16 = P.reshape(_NP16, 16)
    out8 = _sc_accum(dst_pad.reshape(-1, 128), p16)
    out = (out8.reshape(8, _ACCR, 16)[:, :n_out]
           .transpose(1, 0, 2).reshape(n_out, 128))
    return out
